# baseline (speedup 1.0000x reference)
import functools

import jax
import jax.numpy as jnp
from jax import lax
from jax.experimental import pallas as pl
from jax.experimental.pallas import tpu as pltpu
from jax.experimental.pallas import tpu_sc as plsc


def _shuffle_body(t_ref, o_ref):
    o_ref[...] = jnp.transpose(t_ref[...], (0, 2, 1))


def _shuffle_table(table3):
    v1, h, nw = table3.shape
    vb = 91 if v1 % 91 == 0 else v1
    grid = v1 // vb
    return pl.pallas_call(
        _shuffle_body,
        grid=(grid,),
        in_specs=[pl.BlockSpec((vb, h, nw), lambda i: (i, 0, 0))],
        out_specs=pl.BlockSpec((vb, nw, h), lambda i: (i, 0, 0)),
        out_shape=jax.ShapeDtypeStruct((v1, nw, h), jnp.float32),
    )(table3)


def _sc_gather(tsh, tok, chunk=40):
    n, d = tok.shape[0], tsh.shape[1]
    info = plsc.get_sparse_core_info()
    nworkers = info.num_cores * info.num_subcores
    per_w = n // nworkers
    nchunks = per_w // chunk
    assert per_w % chunk == 0 and n % nworkers == 0

    mesh = plsc.VectorSubcoreMesh(core_axis_name="c", subcore_axis_name="s")

    @functools.partial(
        pl.kernel,
        mesh=mesh,
        out_type=jax.ShapeDtypeStruct((n, d), jnp.float32),
        scratch_types=[
            pltpu.VMEM((chunk,), jnp.int32),
            pltpu.VMEM((chunk, d), jnp.float32),
            pltpu.SemaphoreType.DMA,
        ],
        compiler_params=pltpu.CompilerParams(use_tc_tiling_on_sc=False),
    )
    def k(tsh_hbm, tok_hbm, out_hbm, idx_v, rows_v, sem):
        wid = lax.axis_index("s") * info.num_cores + lax.axis_index("c")
        base = wid * per_w
        for c in range(nchunks):
            off = base + c * chunk
            pltpu.sync_copy(tok_hbm.at[pl.ds(off, chunk)], idx_v)
            pltpu.async_copy(tsh_hbm.at[idx_v], rows_v, sem).wait()
            pltpu.sync_copy(rows_v, out_hbm.at[pl.ds(off, chunk)])

    return k(tsh, tok)


def _proj_body(nw, g_ref, w_ref, b_ref, o_ref):
    wmat = w_ref[...]
    bias = b_ref[...]
    for w in range(nw):
        a = g_ref[0, :, w, :]
        o_ref[0, :, w, :] = (
            lax.dot_general(wmat, a, (((1,), (1,)), ((), ())),
                            preferred_element_type=jnp.float32)
            + bias
        )


def _project(g4, W, bias2):
    b, l, nw, h = g4.shape
    out_d = W.shape[0]
    return pl.pallas_call(
        functools.partial(_proj_body, nw),
        grid=(b,),
        in_specs=[
            pl.BlockSpec((1, l, nw, h), lambda i: (i, 0, 0, 0)),
            pl.BlockSpec((out_d, h), lambda i: (0, 0)),
            pl.BlockSpec((out_d, 1), lambda i: (0, 0)),
        ],
        out_specs=pl.BlockSpec((1, out_d, nw, l), lambda i: (i, 0, 0, 0)),
        out_shape=jax.ShapeDtypeStruct((b, out_d, nw, l), jnp.float32),
    )(g4, W, bias2)


def kernel(batchTokens, table, W, b):
    bb, ll = batchTokens.shape
    v1, emb = table.shape
    out_d, hid = W.shape
    nw = emb // hid

    table3 = table.reshape(v1, hid, nw)
    tsh = _shuffle_table(table3).reshape(v1, emb)
    tok = batchTokens.reshape(-1)
    g = _sc_gather(tsh, tok)
    g4 = g.reshape(bb, ll, nw, hid)
    return _project(g4, W, b.reshape(out_d, 1))


# speedup vs baseline: 1.9861x; 1.9861x over previous
import functools

import jax
import jax.numpy as jnp
from jax import lax
from jax.experimental import pallas as pl
from jax.experimental.pallas import tpu as pltpu
from jax.experimental.pallas import tpu_sc as plsc

LANES = 128


def _shuffle_body(hid, nw, dpad, t_ref, o_ref):
    vb = t_ref.shape[0]
    x = t_ref[...]
    xt = jnp.transpose(x.reshape(vb, hid, nw), (0, 2, 1))
    o_ref[:, : hid * nw] = xt.reshape(vb, nw * hid)
    o_ref[:, hid * nw :] = jnp.zeros((vb, dpad - hid * nw), jnp.float32)


def _shuffle_table(table, hid, nw, dpad):
    v1, emb = table.shape
    vb = 96
    return pl.pallas_call(
        functools.partial(_shuffle_body, hid, nw, dpad),
        grid=((v1 + vb - 1) // vb,),
        in_specs=[pl.BlockSpec((vb, emb), lambda i: (i, 0))],
        out_specs=pl.BlockSpec((vb, dpad), lambda i: (i, 0)),
        out_shape=jax.ShapeDtypeStruct((v1, dpad), jnp.float32),
    )(table)


def _sc_gather(tsh, tok, chunk=40):
    n, d = tok.shape[0], tsh.shape[1]
    info = plsc.get_sparse_core_info()
    nworkers = info.num_cores * info.num_subcores
    per_w = n // nworkers
    nchunks = per_w // chunk
    assert per_w % chunk == 0 and n % nworkers == 0

    mesh = plsc.VectorSubcoreMesh(core_axis_name="c", subcore_axis_name="s")

    @functools.partial(
        pl.kernel,
        mesh=mesh,
        out_type=jax.ShapeDtypeStruct((n, d), jnp.float32),
        scratch_types=[
            pltpu.VMEM((chunk,), jnp.int32),
            pltpu.VMEM((chunk, d), jnp.float32),
            pltpu.SemaphoreType.DMA,
        ],
    )
    def k(tsh_hbm, tok_hbm, out_hbm, idx_v, rows_v, sem):
        wid = lax.axis_index("s") * info.num_cores + lax.axis_index("c")
        base = wid * per_w
        for c in range(nchunks):
            off = base + c * chunk
            pltpu.sync_copy(tok_hbm.at[pl.ds(off, chunk)], idx_v)
            pltpu.async_copy(tsh_hbm.at[idx_v], rows_v, sem).wait()
            pltpu.sync_copy(rows_v, out_hbm.at[pl.ds(off, chunk)])

    return k(tsh, tok)


def _proj_body(hid, nw, g_ref, w2_ref, b_ref, o_ref):
    w2 = w2_ref[...]
    bias = b_ref[...]
    npairs = nw // 2
    for p in range(npairs):
        a = g_ref[:, p * 2 * hid : (p + 1) * 2 * hid]
        o2 = lax.dot_general(w2, a, (((1,), (1,)), ((), ())),
                             preferred_element_type=jnp.float32)
        o_ref[0, :, 2 * p, :] = o2[:hid, :] + bias
        o_ref[0, :, 2 * p + 1, :] = o2[hid:, :] + bias
    if nw % 2:
        w = nw - 1
        a = g_ref[:, w * hid : (w + 1) * hid]
        o1 = lax.dot_general(w2[:hid, :hid], a, (((1,), (1,)), ((), ())),
                             preferred_element_type=jnp.float32)
        o_ref[0, :, w, :] = o1 + bias


def _project(g, W2, bias2, bb, ll, nw, hid, out_d, dpad):
    return pl.pallas_call(
        functools.partial(_proj_body, hid, nw),
        grid=(bb,),
        in_specs=[
            pl.BlockSpec((ll, dpad), lambda i: (i, 0)),
            pl.BlockSpec((2 * hid, 2 * hid), lambda i: (0, 0)),
            pl.BlockSpec((out_d, 1), lambda i: (0, 0)),
        ],
        out_specs=pl.BlockSpec((1, out_d, nw, ll), lambda i: (i, 0, 0, 0)),
        out_shape=jax.ShapeDtypeStruct((bb, out_d, nw, ll), jnp.float32),
    )(g, W2, bias2)


def kernel(batchTokens, table, W, b):
    bb, ll = batchTokens.shape
    v1, emb = table.shape
    out_d, hid = W.shape
    nw = emb // hid
    dpad = ((emb + LANES - 1) // LANES) * LANES

    tsh = _shuffle_table(table, hid, nw, dpad)
    tok = batchTokens.reshape(-1)
    g = _sc_gather(tsh, tok)
    z = jnp.zeros((hid, hid), jnp.float32)
    W2 = jnp.block([[W, z], [z, W]])
    return _project(g, W2, b.reshape(out_d, 1), bb, ll, nw, hid, out_d, dpad)


# speedup vs baseline: 2.0231x; 1.0186x over previous
import functools

import jax
import jax.numpy as jnp
from jax import lax
from jax.experimental import pallas as pl
from jax.experimental.pallas import tpu as pltpu
from jax.experimental.pallas import tpu_sc as plsc

LANES = 128


def _shuffle_body(hid, nw, dpad, t_ref, o_ref):
    vb = t_ref.shape[0]
    x = t_ref[...]
    xt = jnp.transpose(x.reshape(vb, hid, nw), (0, 2, 1))
    o_ref[:, : hid * nw] = xt.reshape(vb, nw * hid)
    o_ref[:, hid * nw :] = jnp.zeros((vb, dpad - hid * nw), jnp.float32)


def _shuffle_table(table, hid, nw, dpad):
    v1, emb = table.shape
    vb = 96
    return pl.pallas_call(
        functools.partial(_shuffle_body, hid, nw, dpad),
        grid=((v1 + vb - 1) // vb,),
        in_specs=[pl.BlockSpec((vb, emb), lambda i: (i, 0))],
        out_specs=pl.BlockSpec((vb, dpad), lambda i: (i, 0)),
        out_shape=jax.ShapeDtypeStruct((v1, dpad), jnp.float32),
    )(table)


def _sc_gather(tsh, tok, chunk=40):
    n, d = tok.shape[0], tsh.shape[1]
    info = plsc.get_sparse_core_info()
    nworkers = info.num_cores * info.num_subcores
    nchunks = n // chunk
    nrounds = -(-nchunks // nworkers)
    assert n % chunk == 0 and chunk % 8 == 0

    mesh = plsc.VectorSubcoreMesh(core_axis_name="c", subcore_axis_name="s")

    @functools.partial(
        pl.kernel,
        mesh=mesh,
        out_type=jax.ShapeDtypeStruct((n, d), jnp.float32),
        scratch_types=[
            pltpu.VMEM((chunk,), jnp.int32),
            pltpu.VMEM((chunk, d), jnp.float32),
            pltpu.SemaphoreType.DMA,
        ],
    )
    def k(tsh_hbm, tok_hbm, out_hbm, idx_v, rows_v, sem):
        wid = lax.axis_index("s") * info.num_cores + lax.axis_index("c")
        for r in range(nrounds):
            cid = r * nworkers + wid

            @pl.when(cid < nchunks)
            def _():
                off = cid * chunk
                pltpu.sync_copy(tok_hbm.at[pl.ds(off, chunk)], idx_v)
                pltpu.async_copy(tsh_hbm.at[idx_v], rows_v, sem).wait()
                pltpu.sync_copy(rows_v, out_hbm.at[pl.ds(off, chunk)])

    return k(tsh, tok)


def _proj_body(hid, nw, g_ref, w2_ref, b_ref, *rest):
    o_ref = rest[-1]
    w2 = w2_ref[...]
    bias = b_ref[...]
    npairs = nw // 2
    for p in range(npairs):
        a = g_ref[:, p * 2 * hid : (p + 1) * 2 * hid]
        o2 = lax.dot_general(w2, a, (((1,), (1,)), ((), ())),
                             preferred_element_type=jnp.float32)
        o_ref[0, :, 2 * p, :] = o2[:hid, :] + bias
        o_ref[0, :, 2 * p + 1, :] = o2[hid:, :] + bias
    if nw % 2:
        w = nw - 1
        a = g_ref[:, w * hid : (w + 1) * hid]
        o1 = lax.dot_general(w2[:hid, :hid], a, (((1,), (1,)), ((), ())),
                             preferred_element_type=jnp.float32)
        o_ref[0, :, w, :] = o1 + bias


def _project_slice(g_slice, W2, bias2, prev_out, b0, nb, bb, ll, nw, hid, out_d, dpad):
    in_specs = [
        pl.BlockSpec((ll, dpad), lambda i: (i, 0)),
        pl.BlockSpec((2 * hid, 2 * hid), lambda i: (0, 0)),
        pl.BlockSpec((out_d, 1), lambda i: (0, 0)),
    ]
    args = [g_slice, W2, bias2]
    aliases = {}
    if prev_out is not None:
        in_specs.append(pl.BlockSpec(memory_space=pl.ANY))
        args.append(prev_out)
        aliases = {3: 0}
    return pl.pallas_call(
        functools.partial(_proj_body, hid, nw),
        grid=(nb,),
        in_specs=in_specs,
        out_specs=pl.BlockSpec((1, out_d, nw, ll), lambda i, b0=b0: (b0 + i, 0, 0, 0)),
        out_shape=jax.ShapeDtypeStruct((bb, out_d, nw, ll), jnp.float32),
        input_output_aliases=aliases,
    )(*args)


def kernel(batchTokens, table, W, b):
    bb, ll = batchTokens.shape
    v1, emb = table.shape
    out_d, hid = W.shape
    nw = emb // hid
    dpad = ((emb + LANES - 1) // LANES) * LANES

    tsh = _shuffle_table(table, hid, nw, dpad)
    tok = batchTokens.reshape(-1)
    z = jnp.zeros((hid, hid), jnp.float32)
    W2 = jnp.block([[W, z], [z, W]])
    bias2 = b.reshape(out_d, 1)

    nslices = 4
    nb = bb // nslices
    out = None
    for s in range(nslices):
        g_s = _sc_gather(tsh, tok[s * nb * ll : (s + 1) * nb * ll], chunk=40)
        out = _project_slice(g_s, W2, bias2, out, s * nb, nb, bb, ll, nw,
                             hid, out_d, dpad)
    return out


# speedup vs baseline: 2.3053x; 1.1395x over previous
import functools

import jax
import jax.numpy as jnp
from jax import lax
from jax.experimental import pallas as pl
from jax.experimental.pallas import tpu as pltpu
from jax.experimental.pallas import tpu_sc as plsc

LANES = 128


def _shuffle_body(hid, nw, dpad, t_ref, o_ref):
    vb = t_ref.shape[0]
    half = hid // 2
    x = t_ref[...]
    xt = jnp.transpose(x.reshape(vb, hid, nw), (0, 2, 1))
    lo = lax.bitcast_convert_type(xt[:, :, :half], jnp.uint32)
    hi = lax.bitcast_convert_type(xt[:, :, half:], jnp.uint32)
    lo_r = (lo + 0x7FFF + ((lo >> 16) & 1)) >> 16
    hi_r = (hi + 0x7FFF + ((hi >> 16) & 1)) >> 16
    packed = lo_r | (hi_r << 16)
    pf = lax.bitcast_convert_type(packed, jnp.float32).reshape(vb, nw * half)
    o_ref[:, : nw * half] = pf
    o_ref[:, nw * half :] = jnp.zeros((vb, dpad - nw * half), jnp.float32)


def _shuffle_table(table, hid, nw, dpad):
    v1, emb = table.shape
    vb = 96
    return pl.pallas_call(
        functools.partial(_shuffle_body, hid, nw, dpad),
        grid=((v1 + vb - 1) // vb,),
        in_specs=[pl.BlockSpec((vb, emb), lambda i: (i, 0))],
        out_specs=pl.BlockSpec((vb, dpad), lambda i: (i, 0)),
        out_shape=jax.ShapeDtypeStruct((v1, dpad), jnp.float32),
    )(table)


def _sc_gather(tsh, tok, chunk=80):
    n, d = tok.shape[0], tsh.shape[1]
    info = plsc.get_sparse_core_info()
    nworkers = info.num_cores * info.num_subcores
    nchunks = n // chunk
    nrounds = -(-nchunks // nworkers)
    assert n % chunk == 0 and chunk % 8 == 0

    mesh = plsc.VectorSubcoreMesh(core_axis_name="c", subcore_axis_name="s")

    @functools.partial(
        pl.kernel,
        mesh=mesh,
        out_type=jax.ShapeDtypeStruct((n, d), jnp.float32),
        scratch_types=[
            pltpu.VMEM((chunk,), jnp.int32),
            pltpu.VMEM((chunk, d), jnp.float32),
            pltpu.SemaphoreType.DMA,
        ],
    )
    def k(tsh_hbm, tok_hbm, out_hbm, idx_v, rows_v, sem):
        wid = lax.axis_index("s") * info.num_cores + lax.axis_index("c")
        for r in range(nrounds):
            cid = r * nworkers + wid

            @pl.when(cid < nchunks)
            def _():
                off = cid * chunk
                pltpu.sync_copy(tok_hbm.at[pl.ds(off, chunk)], idx_v)
                pltpu.async_copy(tsh_hbm.at[idx_v], rows_v, sem).wait()
                pltpu.sync_copy(rows_v, out_hbm.at[pl.ds(off, chunk)])

    return k(tsh, tok)


def _proj_body(hid, nw, g_ref, wlo_ref, whi_ref, b_ref, *rest):
    o_ref = rest[-1]
    wlo = wlo_ref[...]
    whi = whi_ref[...]
    bias = b_ref[...]
    out_d = bias.shape[0]
    ngroups = -(-nw // 4)
    for p in range(ngroups):
        u = lax.bitcast_convert_type(g_ref[:, p * 128 : (p + 1) * 128], jnp.uint32)
        a_lo = lax.bitcast_convert_type(u << 16, jnp.float32)
        a_hi = lax.bitcast_convert_type(u & jnp.uint32(0xFFFF0000), jnp.float32)
        o4 = lax.dot_general(wlo, a_lo, (((1,), (1,)), ((), ())),
                             preferred_element_type=jnp.float32)
        o4 = o4 + lax.dot_general(whi, a_hi, (((1,), (1,)), ((), ())),
                                  preferred_element_type=jnp.float32)
        for k in range(min(4, nw - 4 * p)):
            o_ref[0, :, 4 * p + k, :] = o4[k * out_d : (k + 1) * out_d, :] + bias


def _project_slice(g_slice, Wlo4, Whi4, bias2, prev_out, b0, nb, bb, ll, nw,
                   hid, out_d, dpad):
    in_specs = [
        pl.BlockSpec((ll, dpad), lambda i: (i, 0)),
        pl.BlockSpec(Wlo4.shape, lambda i: (0, 0)),
        pl.BlockSpec(Whi4.shape, lambda i: (0, 0)),
        pl.BlockSpec((out_d, 1), lambda i: (0, 0)),
    ]
    args = [g_slice, Wlo4, Whi4, bias2]
    aliases = {}
    if prev_out is not None:
        in_specs.append(pl.BlockSpec(memory_space=pl.ANY))
        args.append(prev_out)
        aliases = {4: 0}
    return pl.pallas_call(
        functools.partial(_proj_body, hid, nw),
        grid=(nb,),
        in_specs=in_specs,
        out_specs=pl.BlockSpec((1, out_d, nw, ll), lambda i, b0=b0: (b0 + i, 0, 0, 0)),
        out_shape=jax.ShapeDtypeStruct((bb, out_d, nw, ll), jnp.float32),
        input_output_aliases=aliases,
    )(*args)


def _blockdiag4(Wh):
    out_d, half = Wh.shape
    z = jnp.zeros((out_d, half), jnp.float32)
    rows = []
    for i in range(4):
        rows.append(jnp.concatenate([Wh if j == i else z for j in range(4)], axis=1))
    return jnp.concatenate(rows, axis=0)


def kernel(batchTokens, table, W, b):
    bb, ll = batchTokens.shape
    v1, emb = table.shape
    out_d, hid = W.shape
    nw = emb // hid
    half = hid // 2
    npk = nw * half
    dpad = ((npk + LANES - 1) // LANES) * LANES

    tsh = _shuffle_table(table, hid, nw, dpad)
    tok = batchTokens.reshape(-1)
    Wlo4 = _blockdiag4(W[:, :half])
    Whi4 = _blockdiag4(W[:, half:])
    bias2 = b.reshape(out_d, 1)

    nslices = 4
    nb = bb // nslices
    out = None
    for s in range(nslices):
        g_s = _sc_gather(tsh, tok[s * nb * ll : (s + 1) * nb * ll], chunk=80)
        out = _project_slice(g_s, Wlo4, Whi4, bias2, out, s * nb, nb, bb, ll,
                             nw, hid, out_d, dpad)
    return out


# speedup vs baseline: 2.3241x; 1.0081x over previous
import functools

import jax
import jax.numpy as jnp
from jax import lax
from jax.experimental import pallas as pl
from jax.experimental.pallas import tpu as pltpu
from jax.experimental.pallas import tpu_sc as plsc

LANES = 128


def _shuffle_body(hid, nw, dpad, t_ref, o_ref):
    vb = t_ref.shape[0]
    half = hid // 2
    x = t_ref[...]
    xt = jnp.transpose(x.reshape(vb, hid, nw), (0, 2, 1))
    lo = lax.bitcast_convert_type(xt[:, :, :half], jnp.uint32)
    hi = lax.bitcast_convert_type(xt[:, :, half:], jnp.uint32)
    lo_r = (lo + 0x7FFF + ((lo >> 16) & 1)) >> 16
    hi_r = (hi + 0x7FFF + ((hi >> 16) & 1)) >> 16
    packed = lo_r | (hi_r << 16)
    pf = lax.bitcast_convert_type(packed, jnp.float32).reshape(vb, nw * half)
    o_ref[:, : nw * half] = pf
    o_ref[:, nw * half :] = jnp.zeros((vb, dpad - nw * half), jnp.float32)


def _shuffle_table(table, hid, nw, dpad):
    v1, emb = table.shape
    vb = 96
    return pl.pallas_call(
        functools.partial(_shuffle_body, hid, nw, dpad),
        grid=((v1 + vb - 1) // vb,),
        in_specs=[pl.BlockSpec((vb, emb), lambda i: (i, 0))],
        out_specs=pl.BlockSpec((vb, dpad), lambda i: (i, 0)),
        out_shape=jax.ShapeDtypeStruct((v1, dpad), jnp.float32),
    )(table)


def _sc_gather(tsh, tok):
    n, d = tok.shape[0], tsh.shape[1]
    info = plsc.get_sparse_core_info()
    nworkers = info.num_cores * info.num_subcores
    span = -(-n // (nworkers * 8)) * 8
    c1 = (span // 2 + 7) // 8 * 8
    c2 = span - c1
    assert n % 8 == 0 and span >= 8 and c2 >= 8

    mesh = plsc.VectorSubcoreMesh(core_axis_name="c", subcore_axis_name="s")

    @functools.partial(
        pl.kernel,
        mesh=mesh,
        out_type=jax.ShapeDtypeStruct((n, d), jnp.float32),
        scratch_types=[
            pltpu.VMEM((c1,), jnp.int32),
            pltpu.VMEM((c2,), jnp.int32),
            pltpu.VMEM((c1, d), jnp.float32),
            pltpu.VMEM((c2, d), jnp.float32),
            pltpu.SemaphoreType.DMA,
            pltpu.SemaphoreType.DMA,
            pltpu.SemaphoreType.DMA,
        ],
    )
    def k(tsh_hbm, tok_hbm, out_hbm, idx1, idx2, buf1, buf2, sg1, sg2, sw):
        wid = lax.axis_index("s") * info.num_cores + lax.axis_index("c")
        start = pl.multiple_of(jnp.minimum(wid * span, n - span), 8)
        pltpu.sync_copy(tok_hbm.at[pl.ds(start, c1)], idx1)
        pltpu.sync_copy(tok_hbm.at[pl.ds(start + c1, c2)], idx2)
        g1 = pltpu.async_copy(tsh_hbm.at[idx1], buf1, sg1)
        g2 = pltpu.async_copy(tsh_hbm.at[idx2], buf2, sg2)
        g1.wait()
        w1 = pltpu.async_copy(buf1, out_hbm.at[pl.ds(start, c1)], sw)
        g2.wait()
        w2 = pltpu.async_copy(buf2, out_hbm.at[pl.ds(start + c1, c2)], sw)
        w1.wait()
        w2.wait()

    return k(tsh, tok)


def _proj_body(hid, nw, g_ref, wlo_ref, whi_ref, b_ref, *rest):
    o_ref = rest[-1]
    wlo = wlo_ref[...]
    whi = whi_ref[...]
    bias = b_ref[...]
    out_d = bias.shape[0]
    ngroups = -(-nw // 4)
    for p in range(ngroups):
        u = lax.bitcast_convert_type(g_ref[:, p * 128 : (p + 1) * 128], jnp.uint32)
        a_lo = lax.bitcast_convert_type(u << 16, jnp.float32)
        a_hi = lax.bitcast_convert_type(u & jnp.uint32(0xFFFF0000), jnp.float32)
        o4 = lax.dot_general(wlo, a_lo, (((1,), (1,)), ((), ())),
                             preferred_element_type=jnp.float32)
        o4 = o4 + lax.dot_general(whi, a_hi, (((1,), (1,)), ((), ())),
                                  preferred_element_type=jnp.float32)
        for k in range(min(4, nw - 4 * p)):
            o_ref[0, :, 4 * p + k, :] = o4[k * out_d : (k + 1) * out_d, :] + bias


def _project_slice(g_slice, Wlo4, Whi4, bias2, prev_out, b0, nb, bb, ll, nw,
                   hid, out_d, dpad):
    in_specs = [
        pl.BlockSpec((ll, dpad), lambda i: (i, 0)),
        pl.BlockSpec(Wlo4.shape, lambda i: (0, 0)),
        pl.BlockSpec(Whi4.shape, lambda i: (0, 0)),
        pl.BlockSpec((out_d, 1), lambda i: (0, 0)),
    ]
    args = [g_slice, Wlo4, Whi4, bias2]
    aliases = {}
    if prev_out is not None:
        in_specs.append(pl.BlockSpec(memory_space=pl.ANY))
        args.append(prev_out)
        aliases = {4: 0}
    return pl.pallas_call(
        functools.partial(_proj_body, hid, nw),
        grid=(nb,),
        in_specs=in_specs,
        out_specs=pl.BlockSpec((1, out_d, nw, ll), lambda i, b0=b0: (b0 + i, 0, 0, 0)),
        out_shape=jax.ShapeDtypeStruct((bb, out_d, nw, ll), jnp.float32),
        input_output_aliases=aliases,
    )(*args)


def _blockdiag4(Wh):
    out_d, half = Wh.shape
    z = jnp.zeros((out_d, half), jnp.float32)
    rows = []
    for i in range(4):
        rows.append(jnp.concatenate([Wh if j == i else z for j in range(4)], axis=1))
    return jnp.concatenate(rows, axis=0)


def kernel(batchTokens, table, W, b):
    bb, ll = batchTokens.shape
    v1, emb = table.shape
    out_d, hid = W.shape
    nw = emb // hid
    half = hid // 2
    npk = nw * half
    dpad = ((npk + LANES - 1) // LANES) * LANES

    tsh = _shuffle_table(table, hid, nw, dpad)
    tok = batchTokens.reshape(-1)
    Wlo4 = _blockdiag4(W[:, :half])
    Whi4 = _blockdiag4(W[:, half:])
    bias2 = b.reshape(out_d, 1)

    nslices = 4
    nb = bb // nslices
    out = None
    for s in range(nslices):
        g_s = _sc_gather(tsh, tok[s * nb * ll : (s + 1) * nb * ll])
        out = _project_slice(g_s, Wlo4, Whi4, bias2, out, s * nb, nb, bb, ll,
                             nw, hid, out_d, dpad)
    return out


# speedup vs baseline: 3.6549x; 1.5726x over previous
import functools

import jax
import jax.numpy as jnp
from jax import lax
from jax.experimental import pallas as pl
from jax.experimental.pallas import tpu as pltpu
from jax.experimental.pallas import tpu_sc as plsc

LANES = 128


def _shuffle_body(hid, nw, dpad, t_ref, o_ref):
    vb = t_ref.shape[0]
    half = hid // 2
    x = t_ref[...]
    xt = jnp.transpose(x.reshape(vb, hid, nw), (0, 2, 1))
    lo = lax.bitcast_convert_type(xt[:, :, :half], jnp.uint32)
    hi = lax.bitcast_convert_type(xt[:, :, half:], jnp.uint32)
    lo_r = (lo + 0x7FFF + ((lo >> 16) & 1)) >> 16
    hi_r = (hi + 0x7FFF + ((hi >> 16) & 1)) >> 16
    packed = lo_r | (hi_r << 16)
    pf = lax.bitcast_convert_type(packed, jnp.float32).reshape(vb, nw * half)
    o_ref[:, : nw * half] = pf
    o_ref[:, nw * half :] = jnp.zeros((vb, dpad - nw * half), jnp.float32)


def _shuffle_table(table, hid, nw, dpad):
    v1, emb = table.shape
    vb = 96
    return pl.pallas_call(
        functools.partial(_shuffle_body, hid, nw, dpad),
        grid=((v1 + vb - 1) // vb,),
        in_specs=[pl.BlockSpec((vb, emb), lambda i: (i, 0))],
        out_specs=pl.BlockSpec((vb, dpad), lambda i: (i, 0)),
        out_shape=jax.ShapeDtypeStruct((v1, dpad), jnp.float32),
    )(table)


def _sc_gather(tsh, tok):
    n, d = tok.shape[0], tsh.shape[1]
    info = plsc.get_sparse_core_info()
    nworkers = info.num_cores * info.num_subcores
    span = -(-n // (nworkers * 8)) * 8
    c1 = (span // 2 + 7) // 8 * 8
    c2 = span - c1
    assert n % 8 == 0 and span >= 8 and c2 >= 8

    mesh = plsc.VectorSubcoreMesh(core_axis_name="c", subcore_axis_name="s")

    @functools.partial(
        pl.kernel,
        mesh=mesh,
        out_type=jax.ShapeDtypeStruct((n, d), jnp.float32),
        scratch_types=[
            pltpu.VMEM((c1,), jnp.int32),
            pltpu.VMEM((c2,), jnp.int32),
            pltpu.VMEM((c1, d), jnp.float32),
            pltpu.VMEM((c2, d), jnp.float32),
            pltpu.SemaphoreType.DMA,
            pltpu.SemaphoreType.DMA,
            pltpu.SemaphoreType.DMA,
        ],
    )
    def k(tsh_hbm, tok_hbm, out_hbm, idx1, idx2, buf1, buf2, sg1, sg2, sw):
        wid = lax.axis_index("s") * info.num_cores + lax.axis_index("c")
        start = pl.multiple_of(jnp.minimum(wid * span, n - span), 8)
        pltpu.sync_copy(tok_hbm.at[pl.ds(start, c1)], idx1)
        pltpu.sync_copy(tok_hbm.at[pl.ds(start + c1, c2)], idx2)
        g1 = pltpu.async_copy(tsh_hbm.at[idx1], buf1, sg1)
        g2 = pltpu.async_copy(tsh_hbm.at[idx2], buf2, sg2)
        g1.wait()
        w1 = pltpu.async_copy(buf1, out_hbm.at[pl.ds(start, c1)], sw)
        g2.wait()
        w2 = pltpu.async_copy(buf2, out_hbm.at[pl.ds(start + c1, c2)], sw)
        w1.wait()
        w2.wait()

    return k(tsh, tok)


def _proj_body(hid, nw, g_ref, wlo_ref, whi_ref, b_ref, *rest):
    o_ref = rest[-1]
    wlo = wlo_ref[...]
    whi = whi_ref[...]
    bias = b_ref[...]
    out_d = bias.shape[0]
    ngroups = -(-nw // 4)
    for p in range(ngroups):
        u = lax.bitcast_convert_type(g_ref[:, p * 128 : (p + 1) * 128], jnp.uint32)
        a_lo = lax.bitcast_convert_type(u << 16, jnp.float32)
        a_hi = lax.bitcast_convert_type(u & jnp.uint32(0xFFFF0000), jnp.float32)
        o4 = lax.dot_general(wlo, a_lo, (((1,), (1,)), ((), ())),
                             preferred_element_type=jnp.float32)
        o4 = o4 + lax.dot_general(whi, a_hi, (((1,), (1,)), ((), ())),
                                  preferred_element_type=jnp.float32)
        for k in range(min(4, nw - 4 * p)):
            o_ref[0, 4 * p + k, :, :] = o4[k * out_d : (k + 1) * out_d, :] + bias


def _project_slice(g_slice, Wlo4, Whi4, bias2, prev_out, b0, nb, bb, ll, nw,
                   hid, out_d, dpad):
    in_specs = [
        pl.BlockSpec((ll, dpad), lambda i: (i, 0)),
        pl.BlockSpec(Wlo4.shape, lambda i: (0, 0)),
        pl.BlockSpec(Whi4.shape, lambda i: (0, 0)),
        pl.BlockSpec((out_d, 1), lambda i: (0, 0)),
    ]
    args = [g_slice, Wlo4, Whi4, bias2]
    aliases = {}
    if prev_out is not None:
        in_specs.append(pl.BlockSpec(memory_space=pl.ANY))
        args.append(prev_out)
        aliases = {4: 0}
    return pl.pallas_call(
        functools.partial(_proj_body, hid, nw),
        grid=(nb,),
        in_specs=in_specs,
        out_specs=pl.BlockSpec((1, nw, out_d, ll), lambda i, b0=b0: (b0 + i, 0, 0, 0)),
        out_shape=jax.ShapeDtypeStruct((bb, nw, out_d, ll), jnp.float32),
        input_output_aliases=aliases,
    )(*args)


def _blockdiag4(Wh):
    out_d, half = Wh.shape
    z = jnp.zeros((out_d, half), jnp.float32)
    rows = []
    for i in range(4):
        rows.append(jnp.concatenate([Wh if j == i else z for j in range(4)], axis=1))
    return jnp.concatenate(rows, axis=0)


def kernel(batchTokens, table, W, b):
    bb, ll = batchTokens.shape
    v1, emb = table.shape
    out_d, hid = W.shape
    nw = emb // hid
    half = hid // 2
    npk = nw * half
    dpad = ((npk + LANES - 1) // LANES) * LANES

    tsh = _shuffle_table(table, hid, nw, dpad)
    tok = batchTokens.reshape(-1)
    Wlo4 = _blockdiag4(W[:, :half])
    Whi4 = _blockdiag4(W[:, half:])
    bias2 = b.reshape(out_d, 1)

    nslices = 4
    nb = bb // nslices
    out = None
    for s in range(nslices):
        g_s = _sc_gather(tsh, tok[s * nb * ll : (s + 1) * nb * ll])
        out = _project_slice(g_s, Wlo4, Whi4, bias2, out, s * nb, nb, bb, ll,
                             nw, hid, out_d, dpad)
    return jnp.transpose(out, (0, 2, 1, 3))


# speedup vs baseline: 3.8475x; 1.0527x over previous
import functools

import jax
import jax.numpy as jnp
from jax import lax
from jax.experimental import pallas as pl
from jax.experimental.pallas import tpu as pltpu
from jax.experimental.pallas import tpu_sc as plsc

LANES = 128


def _shuffle_body(hid, nw, dpad, t_ref, o_ref):
    vb = t_ref.shape[0]
    half = hid // 2
    npk = nw * half
    x = t_ref[...]
    lo = lax.bitcast_convert_type(x[:, :npk], jnp.uint32)
    hi = lax.bitcast_convert_type(x[:, npk : 2 * npk], jnp.uint32)
    lo_r = (lo + 0x7FFF + ((lo >> 16) & 1)) >> 16
    hi_r = (hi + 0x7FFF + ((hi >> 16) & 1)) >> 16
    packed = (lo_r | (hi_r << 16)).reshape(vb, half, nw)
    pt = jnp.transpose(packed, (0, 2, 1)).reshape(vb, npk)
    o_ref[:, :npk] = lax.bitcast_convert_type(pt, jnp.float32)
    o_ref[:, npk:] = jnp.zeros((vb, dpad - npk), jnp.float32)


def _shuffle_table(table, hid, nw, dpad):
    v1, emb = table.shape
    vb = 96
    return pl.pallas_call(
        functools.partial(_shuffle_body, hid, nw, dpad),
        grid=((v1 + vb - 1) // vb,),
        in_specs=[pl.BlockSpec((vb, emb), lambda i: (i, 0))],
        out_specs=pl.BlockSpec((vb, dpad), lambda i: (i, 0)),
        out_shape=jax.ShapeDtypeStruct((v1, dpad), jnp.float32),
    )(table)


def _sc_gather(tsh, tok):
    n, d = tok.shape[0], tsh.shape[1]
    info = plsc.get_sparse_core_info()
    nworkers = info.num_cores * info.num_subcores
    span = -(-n // (nworkers * 8)) * 8
    c1 = (span // 2 + 7) // 8 * 8
    c2 = span - c1
    assert n % 8 == 0 and span >= 8 and c2 >= 8

    mesh = plsc.VectorSubcoreMesh(core_axis_name="c", subcore_axis_name="s")

    @functools.partial(
        pl.kernel,
        mesh=mesh,
        out_type=jax.ShapeDtypeStruct((n, d), jnp.float32),
        scratch_types=[
            pltpu.VMEM((c1,), jnp.int32),
            pltpu.VMEM((c2,), jnp.int32),
            pltpu.VMEM((c1, d), jnp.float32),
            pltpu.VMEM((c2, d), jnp.float32),
            pltpu.SemaphoreType.DMA,
            pltpu.SemaphoreType.DMA,
            pltpu.SemaphoreType.DMA,
        ],
    )
    def k(tsh_hbm, tok_hbm, out_hbm, idx1, idx2, buf1, buf2, sg1, sg2, sw):
        wid = lax.axis_index("s") * info.num_cores + lax.axis_index("c")
        start = pl.multiple_of(jnp.minimum(wid * span, n - span), 8)
        pltpu.sync_copy(tok_hbm.at[pl.ds(start, c1)], idx1)
        pltpu.sync_copy(tok_hbm.at[pl.ds(start + c1, c2)], idx2)
        g1 = pltpu.async_copy(tsh_hbm.at[idx1], buf1, sg1)
        g2 = pltpu.async_copy(tsh_hbm.at[idx2], buf2, sg2)
        g1.wait()
        w1 = pltpu.async_copy(buf1, out_hbm.at[pl.ds(start, c1)], sw)
        g2.wait()
        w2 = pltpu.async_copy(buf2, out_hbm.at[pl.ds(start + c1, c2)], sw)
        w1.wait()
        w2.wait()

    return k(tsh, tok)


def _proj_body(hid, nw, g_ref, wlo_ref, whi_ref, b_ref, *rest):
    o_ref = rest[-1]
    wlo = wlo_ref[...]
    whi = whi_ref[...]
    bias = b_ref[...]
    out_d = bias.shape[0]
    ngroups = -(-nw // 4)
    for p in range(ngroups):
        u = lax.bitcast_convert_type(g_ref[:, p * 128 : (p + 1) * 128], jnp.uint32)
        a_lo = lax.bitcast_convert_type(u << 16, jnp.float32)
        a_hi = lax.bitcast_convert_type(u & jnp.uint32(0xFFFF0000), jnp.float32)
        o4 = lax.dot_general(wlo, a_lo, (((1,), (1,)), ((), ())),
                             preferred_element_type=jnp.float32)
        o4 = o4 + lax.dot_general(whi, a_hi, (((1,), (1,)), ((), ())),
                                  preferred_element_type=jnp.float32)
        for k in range(min(4, nw - 4 * p)):
            o_ref[0, 4 * p + k, :, :] = o4[k * out_d : (k + 1) * out_d, :] + bias


def _project_slice(g_slice, Wlo4, Whi4, bias2, prev_out, b0, nb, bb, ll, nw,
                   hid, out_d, dpad):
    in_specs = [
        pl.BlockSpec((ll, dpad), lambda i: (i, 0)),
        pl.BlockSpec(Wlo4.shape, lambda i: (0, 0)),
        pl.BlockSpec(Whi4.shape, lambda i: (0, 0)),
        pl.BlockSpec((out_d, 1), lambda i: (0, 0)),
    ]
    args = [g_slice, Wlo4, Whi4, bias2]
    aliases = {}
    if prev_out is not None:
        in_specs.append(pl.BlockSpec(memory_space=pl.ANY))
        args.append(prev_out)
        aliases = {4: 0}
    return pl.pallas_call(
        functools.partial(_proj_body, hid, nw),
        grid=(nb,),
        in_specs=in_specs,
        out_specs=pl.BlockSpec((1, nw, out_d, ll), lambda i, b0=b0: (b0 + i, 0, 0, 0)),
        out_shape=jax.ShapeDtypeStruct((bb, nw, out_d, ll), jnp.float32),
        input_output_aliases=aliases,
    )(*args)


def _blockdiag4(Wh):
    out_d, half = Wh.shape
    z = jnp.zeros((out_d, half), jnp.float32)
    rows = []
    for i in range(4):
        rows.append(jnp.concatenate([Wh if j == i else z for j in range(4)], axis=1))
    return jnp.concatenate(rows, axis=0)


def kernel(batchTokens, table, W, b):
    bb, ll = batchTokens.shape
    v1, emb = table.shape
    out_d, hid = W.shape
    nw = emb // hid
    half = hid // 2
    npk = nw * half
    dpad = ((npk + LANES - 1) // LANES) * LANES

    tsh = _shuffle_table(table, hid, nw, dpad)
    tok = batchTokens.reshape(-1)
    Wlo4 = _blockdiag4(W[:, :half])
    Whi4 = _blockdiag4(W[:, half:])
    bias2 = b.reshape(out_d, 1)

    slice_nb = [bb // 8, bb // 8, bb // 4, bb // 4, bb // 4]
    out = None
    b0 = 0
    for nb in slice_nb:
        g_s = _sc_gather(tsh, tok[b0 * ll : (b0 + nb) * ll])
        out = _project_slice(g_s, Wlo4, Whi4, bias2, out, b0, nb, bb, ll,
                             nw, hid, out_d, dpad)
        b0 += nb
    return jnp.transpose(out, (0, 2, 1, 3))
